# baseline (speedup 1.0000x reference)
import functools

import jax
import jax.numpy as jnp
from jax import lax
from jax.experimental import pallas as pl
from jax.experimental.pallas import tpu as pltpu
from jax.experimental.pallas import tpu_sc as plsc

_VOCAB = 1_000_000
_EMB = 32

_NC, _NS = 2, 16
_NW = _NC * _NS
_T = 200
_BATCH = 4096
_TPW = 8
_BPW = 4
_TBLK = _T // _TPW
_BBLK = _BATCH // _BPW
_NPAIR = _TBLK // 2


def _transpose_rows_to_slab(rows_ref, slab_ref):
    iota = lax.iota(jnp.int32, 16)

    def body(i, carry):
        et = i // 8
        btl = i % 8
        for es in range(8):
            cvec = jnp.full((16,), et * 8 + es, jnp.int32)
            for blk in range(8):
                bvec = btl * 128 + blk * 16 + iota
                vals = plsc.load_gather(rows_ref, [bvec, cvec])
                slab_ref[et, btl, es, pl.ds(blk * 16, 16)] = vals
        return carry

    lax.fori_loop(0, 32, body, 0)


def _make_sc_kernel():
    mesh = plsc.VectorSubcoreMesh(core_axis_name="c", subcore_axis_name="s")

    @functools.partial(
        pl.kernel,
        mesh=mesh,
        out_type=jax.ShapeDtypeStruct((_T, 4, 32, 8, 128), jnp.float32),
        scratch_types=[
            pltpu.VMEM((_TBLK, _BBLK), jnp.int32),
            pltpu.VMEM((_BBLK, _EMB), jnp.float32),
            pltpu.VMEM((4, 8, 8, 128), jnp.float32),
            pltpu.VMEM((4, 8, 8, 128), jnp.float32),
            pltpu.SemaphoreType.DMA,
            pltpu.SemaphoreType.DMA,
            pltpu.SemaphoreType.DMA,
        ],
        compiler_params=pltpu.CompilerParams(
            use_tc_tiling_on_sc=False, needs_layout_passes=False),
    )
    def emb_kernel(idsT_hbm, table_hbm, out_hbm, idx_v, rows_v, sa, sb,
                   gsem, oa, ob):
        slab = [sa, sb]
        osem = [oa, ob]
        wid = lax.axis_index("s") * _NC + lax.axis_index("c")
        tw = wid // _BPW
        bw = wid % _BPW
        t0 = tw * _TBLK
        b0 = bw * _BBLK
        bt0 = bw * 8

        pltpu.sync_copy(
            idsT_hbm.at[pl.ds(t0, _TBLK), pl.ds(b0, _BBLK)], idx_v)

        def start_gather(tok):
            return pltpu.async_copy(
                table_hbm.at[idx_v.at[tok]], rows_v, gsem)

        def wait_gather():
            pltpu.make_async_copy(
                table_hbm.at[idx_v.at[0]], rows_v, gsem).wait()

        def start_wb(tok, p):
            for et in range(4):
                pltpu.async_copy(
                    slab[p].at[et],
                    out_hbm.at[t0 + tok, et, pl.ds(bt0, 8)], osem[p])

        def wait_wb(p):
            for et in range(4):
                pltpu.make_async_copy(
                    slab[p].at[et], out_hbm.at[0, et, pl.ds(bt0, 8)],
                    osem[p]).wait()

        start_gather(0)

        def pair(jj, carry):
            for half in range(2):
                tok = 2 * jj + half
                p = half
                wait_gather()

                @pl.when(jj >= 1)
                def _():
                    wait_wb(p)

                _transpose_rows_to_slab(rows_v, slab[p])
                start_gather(tok + 1)
                start_wb(tok, p)
            return carry

        lax.fori_loop(0, _NPAIR, pair, 0)

        wait_gather()
        wait_wb(0)
        _transpose_rows_to_slab(rows_v, slab[0])
        start_wb(_TBLK - 1, 0)
        wait_wb(0)
        wait_wb(1)

    return emb_kernel


def kernel(input_ids, weight):
    idsT4 = (input_ids.astype(jnp.int32) * 4).T
    w4 = jnp.pad(weight, ((0, 0), (0, 3 * _EMB))).reshape(4 * _VOCAB, _EMB)
    buf = _make_sc_kernel()(idsT4, w4)
    return buf.transpose(2, 4, 0, 1, 3).reshape(_BATCH, _T, _EMB)


# speedup vs baseline: 1.5272x; 1.5272x over previous
import functools

import jax
import jax.numpy as jnp
from jax import lax
from jax.experimental import pallas as pl
from jax.experimental.pallas import tpu as pltpu
from jax.experimental.pallas import tpu_sc as plsc

_VOCAB = 1_000_000
_EMB = 32

_NC, _NS = 2, 16
_NW = _NC * _NS
_T = 200
_BATCH = 4096
_TPW = 8
_BPW = 4
_TBLK = _T // _TPW
_BBLK = _BATCH // _BPW
_NPAIR = _TBLK // 2


def _transpose_rows_to_slab(rows_ref, slab_ref):
    iota = lax.iota(jnp.int32, 16)
    perms = [(iota + d) & 15 for d in range(16)]
    perms_hi = [p >> 3 for p in perms]
    perms_lo = [p & 7 for p in perms]

    def body(i, carry):
        bblk = i // 2
        ch = i % 2
        bvec = bblk * 16 + iota
        btl_v = jnp.full((16,), bblk // 8, jnp.int32)
        bl_v = (bblk % 8) * 16 + iota
        for d in range(16):
            cvec = ch * 16 + perms[d]
            et_v = ch * 2 + perms_hi[d]
            vals = plsc.load_gather(rows_ref, [bvec, cvec])
            plsc.store_scatter(
                slab_ref, [et_v, btl_v, perms_lo[d], bl_v], vals)
        return carry

    lax.fori_loop(0, 128, body, 0)


def _make_sc_kernel():
    mesh = plsc.VectorSubcoreMesh(core_axis_name="c", subcore_axis_name="s")

    @functools.partial(
        pl.kernel,
        mesh=mesh,
        out_type=jax.ShapeDtypeStruct((_T, 4, 32, 8, 128), jnp.float32),
        scratch_types=[
            pltpu.VMEM((_TBLK, _BBLK), jnp.int32),
            pltpu.VMEM((_BBLK, _EMB), jnp.float32),
            pltpu.VMEM((4, 8, 8, 128), jnp.float32),
            pltpu.VMEM((4, 8, 8, 128), jnp.float32),
            pltpu.SemaphoreType.DMA,
            pltpu.SemaphoreType.DMA,
            pltpu.SemaphoreType.DMA,
        ],
        compiler_params=pltpu.CompilerParams(
            use_tc_tiling_on_sc=False, needs_layout_passes=False),
    )
    def emb_kernel(idsT_hbm, table_hbm, out_hbm, idx_v, rows_v, sa, sb,
                   gsem, oa, ob):
        slab = [sa, sb]
        osem = [oa, ob]
        wid = lax.axis_index("s") * _NC + lax.axis_index("c")
        tw = wid // _BPW
        bw = wid % _BPW
        t0 = tw * _TBLK
        b0 = bw * _BBLK
        bt0 = bw * 8

        pltpu.sync_copy(
            idsT_hbm.at[pl.ds(t0, _TBLK), pl.ds(b0, _BBLK)], idx_v)

        def start_gather(tok):
            return pltpu.async_copy(
                table_hbm.at[idx_v.at[tok]], rows_v, gsem)

        def wait_gather():
            pltpu.make_async_copy(
                table_hbm.at[idx_v.at[0]], rows_v, gsem).wait()

        def start_wb(tok, p):
            for et in range(4):
                pltpu.async_copy(
                    slab[p].at[et],
                    out_hbm.at[t0 + tok, et, pl.ds(bt0, 8)], osem[p])

        def wait_wb(p):
            for et in range(4):
                pltpu.make_async_copy(
                    slab[p].at[et], out_hbm.at[0, et, pl.ds(bt0, 8)],
                    osem[p]).wait()

        start_gather(0)

        def pair(jj, carry):
            for half in range(2):
                tok = 2 * jj + half
                p = half
                wait_gather()

                @pl.when(jj >= 1)
                def _():
                    wait_wb(p)

                _transpose_rows_to_slab(rows_v, slab[p])
                start_gather(tok + 1)
                start_wb(tok, p)
            return carry

        lax.fori_loop(0, _NPAIR, pair, 0)

        wait_gather()
        wait_wb(0)
        _transpose_rows_to_slab(rows_v, slab[0])
        start_wb(_TBLK - 1, 0)
        wait_wb(0)
        wait_wb(1)

    return emb_kernel


def kernel(input_ids, weight):
    idsT4 = (input_ids.astype(jnp.int32) * 4).T
    w4 = jnp.pad(weight, ((0, 0), (0, 3 * _EMB))).reshape(4 * _VOCAB, _EMB)
    buf = _make_sc_kernel()(idsT4, w4)
    return buf.transpose(2, 4, 0, 1, 3).reshape(_BATCH, _T, _EMB)


# speedup vs baseline: 1.6850x; 1.1033x over previous
import functools

import jax
import jax.numpy as jnp
from jax import lax
from jax.experimental import pallas as pl
from jax.experimental.pallas import tpu as pltpu
from jax.experimental.pallas import tpu_sc as plsc

_VOCAB = 1_000_000
_EMB = 32

_NC, _NS = 2, 16
_NW = _NC * _NS
_T = 200
_BATCH = 4096
_TPW = 8
_BPW = 4
_TBLK = _T // _TPW
_BBLK = _BATCH // _BPW
_H = 512
_NSTEP = _TBLK * 2


def _transpose_rows_to_slab(rows_ref, slab_ref):
    iota = lax.iota(jnp.int32, 16)

    def body(i, carry):
        bblk = i // 2
        ch = i % 2
        bvec = bblk * 16 + iota
        btl_v = jnp.full((16,), bblk // 8, jnp.int32)
        bl_v = (bblk % 8) * 16 + iota
        for d in range(16):
            perm = (iota + d) & 15
            cvec = ch * 16 + perm
            et_v = ch * 2 + (perm >> 3)
            es_v = perm & 7
            vals = plsc.load_gather(rows_ref, [bvec, cvec])
            plsc.store_scatter(slab_ref, [et_v, btl_v, es_v, bl_v], vals)
        return carry

    lax.fori_loop(0, 64, body, 0)


def _make_sc_kernel():
    mesh = plsc.VectorSubcoreMesh(core_axis_name="c", subcore_axis_name="s")

    @functools.partial(
        pl.kernel,
        mesh=mesh,
        out_type=jax.ShapeDtypeStruct((_T, 4, 32, 8, 128), jnp.float32),
        scratch_types=[
            pltpu.VMEM((_TBLK, _BBLK), jnp.int32),
            pltpu.VMEM((_H, _EMB), jnp.float32),
            pltpu.VMEM((_H, _EMB), jnp.float32),
            pltpu.VMEM((4, 4, 8, 128), jnp.float32),
            pltpu.VMEM((4, 4, 8, 128), jnp.float32),
            pltpu.SemaphoreType.DMA,
            pltpu.SemaphoreType.DMA,
            pltpu.SemaphoreType.DMA,
            pltpu.SemaphoreType.DMA,
        ],
        compiler_params=pltpu.CompilerParams(
            use_tc_tiling_on_sc=False, needs_layout_passes=False),
    )
    def emb_kernel(idsT_hbm, table_hbm, out_hbm, idx_v, ra, rb, sa, sb,
                   ga, gb, oa, ob):
        rows = [ra, rb]
        slab = [sa, sb]
        gsem = [ga, gb]
        osem = [oa, ob]
        wid = lax.axis_index("s") * _NC + lax.axis_index("c")
        tw = wid // _BPW
        bw = wid % _BPW
        t0 = tw * _TBLK
        b0 = bw * _BBLK
        bt0 = bw * 8

        pltpu.sync_copy(
            idsT_hbm.at[pl.ds(t0, _TBLK), pl.ds(b0, _BBLK)], idx_v)

        def idx_slice(step):
            tok = step // 2
            half = step % 2
            return idx_v.at[tok, pl.ds(half * _H, _H)]

        def start_gather(step, p):
            pltpu.async_copy(table_hbm.at[idx_slice(step)], rows[p], gsem[p])

        def wait_gather(p):
            pltpu.make_async_copy(
                table_hbm.at[idx_slice(0)], rows[p], gsem[p]).wait()

        def wb_dst(step, et):
            tok = step // 2
            half = step % 2
            return out_hbm.at[t0 + tok, et, pl.ds(bt0 + half * 4, 4)]

        def start_wb(step, p):
            for et in range(4):
                pltpu.async_copy(slab[p].at[et], wb_dst(step, et), osem[p])

        def wait_wb(p):
            for et in range(4):
                pltpu.make_async_copy(
                    slab[p].at[et], wb_dst(0, et), osem[p]).wait()

        start_gather(0, 0)
        start_gather(1, 1)

        def pair(jj, carry):
            for p in range(2):
                step = 2 * jj + p
                wait_gather(p)

                @pl.when(jj >= 1)
                def _():
                    wait_wb(p)

                _transpose_rows_to_slab(rows[p], slab[p])

                @pl.when(step + 2 < _NSTEP)
                def _():
                    start_gather(step + 2, p)
                start_wb(step, p)
            return carry

        lax.fori_loop(0, _NSTEP // 2, pair, 0)
        wait_wb(0)
        wait_wb(1)

    return emb_kernel


def kernel(input_ids, weight):
    idsT4 = (input_ids.astype(jnp.int32) * 4).T
    w4 = jnp.pad(weight, ((0, 0), (0, 3 * _EMB))).reshape(4 * _VOCAB, _EMB)
    buf = _make_sc_kernel()(idsT4, w4)
    return buf.transpose(2, 4, 0, 1, 3).reshape(_BATCH, _T, _EMB)


# speedup vs baseline: 2.3962x; 1.4221x over previous
import functools

import jax
import jax.numpy as jnp
from jax import lax
from jax.experimental import pallas as pl
from jax.experimental.pallas import tpu as pltpu
from jax.experimental.pallas import tpu_sc as plsc

_VOCAB = 1_000_000
_EMB = 32

_NC, _NS = 2, 16
_NW = _NC * _NS
_T = 200
_BATCH = 4096
_TPW = 8
_BPW = 4
_TBLK = _T // _TPW
_BBLK = _BATCH // _BPW
_H = 512
_NSTEP = _TBLK * 2

_NTC = _VOCAB // 128
_MAIN = 244


def _tile_transpose(t4, r1d, nkblk):
    iota = lax.iota(jnp.int32, 16)

    def body(i, carry):
        kv = (i // 2) * 16 + iota
        eh = i % 2
        kv32 = kv * 32
        for d in range(16):
            perm = (iota + d) & 15
            e = eh * 16 + perm
            vals = plsc.load_gather(t4, [e >> 3, e & 7, kv])
            plsc.store_scatter(r1d, [kv32 + e], vals)
        return carry

    lax.fori_loop(0, nkblk * 2, body, 0)


def _make_prep_kernel():
    mesh = plsc.VectorSubcoreMesh(core_axis_name="c", subcore_axis_name="s")

    @functools.partial(
        pl.kernel,
        mesh=mesh,
        out_type=jax.ShapeDtypeStruct((_VOCAB * _EMB,), jnp.float32),
        scratch_types=[
            pltpu.VMEM((4, 8, 128), jnp.float32),
            pltpu.VMEM((4, 8, 128), jnp.float32),
            pltpu.VMEM((4096,), jnp.float32),
            pltpu.VMEM((4096,), jnp.float32),
            pltpu.SemaphoreType.DMA,
            pltpu.SemaphoreType.DMA,
            pltpu.SemaphoreType.DMA,
            pltpu.SemaphoreType.DMA,
        ],
        compiler_params=pltpu.CompilerParams(
            use_tc_tiling_on_sc=True, needs_layout_passes=False),
    )
    def prep_kernel(wt_hbm, tail_hbm, out_hbm, ta, tb, ra, rb, la, lb, oa, ob):
        tiles = [ta, tb]
        rowb = [ra, rb]
        lsem = [la, lb]
        osem = [oa, ob]
        wid = lax.axis_index("s") * _NC + lax.axis_index("c")

        def load(j, p):
            c = wid + _NW * j
            for ct in range(4):
                pltpu.async_copy(
                    wt_hbm.at[pl.ds(ct * 8, 8), pl.ds(c * 128, 128)],
                    tiles[p].at[ct], lsem[p])

        def wait_load(p):
            for ct in range(4):
                pltpu.make_async_copy(
                    wt_hbm.at[pl.ds(0, 8), pl.ds(0, 128)],
                    tiles[p].at[ct], lsem[p]).wait()

        def wb(j, p):
            c = wid + _NW * j
            pltpu.async_copy(
                rowb[p], out_hbm.at[pl.ds(c * 4096, 4096)], osem[p])

        def wait_wb(p):
            pltpu.make_async_copy(
                rowb[p], out_hbm.at[pl.ds(0, 4096)], osem[p]).wait()

        load(0, 0)
        load(1, 1)

        def pair(jj, carry):
            for p in range(2):
                j = 2 * jj + p
                wait_load(p)

                @pl.when(jj >= 1)
                def _():
                    wait_wb(p)

                _tile_transpose(tiles[p], rowb[p], 8)

                @pl.when(j + 2 < _MAIN)
                def _():
                    load(j + 2, p)
                wb(j, p)
            return carry

        lax.fori_loop(0, _MAIN // 2, pair, 0)
        wait_wb(0)
        wait_wb(1)

        @pl.when(wid < 4)
        def _():
            c = _NW * _MAIN + wid
            for ct in range(4):
                pltpu.sync_copy(
                    wt_hbm.at[pl.ds(ct * 8, 8), pl.ds(c * 128, 128)],
                    ta.at[ct])
            _tile_transpose(ta, ra, 8)
            pltpu.sync_copy(ra, out_hbm.at[pl.ds(c * 4096, 4096)])

        @pl.when(wid == 4)
        def _():
            pltpu.sync_copy(tail_hbm, ra.at[pl.ds(0, 2048)])
            pltpu.sync_copy(
                ra.at[pl.ds(0, 2048)],
                out_hbm.at[pl.ds(_NTC * 4096, 2048)])

    return prep_kernel


def _transpose_rows_to_slab(rows_ref, slab_ref):
    iota = lax.iota(jnp.int32, 16)

    def body(i, carry):
        bblk = i // 2
        ch = i % 2
        bvec = bblk * 16 + iota
        btl_v = jnp.full((16,), bblk // 8, jnp.int32)
        bl_v = (bblk % 8) * 16 + iota
        for d in range(16):
            perm = (iota + d) & 15
            cvec = ch * 16 + perm
            et_v = ch * 2 + (perm >> 3)
            es_v = perm & 7
            vals = plsc.load_gather(rows_ref, [bvec, cvec])
            plsc.store_scatter(slab_ref, [et_v, btl_v, es_v, bl_v], vals)
        return carry

    lax.fori_loop(0, 64, body, 0)


def _make_gather_kernel():
    mesh = plsc.VectorSubcoreMesh(core_axis_name="c", subcore_axis_name="s")

    @functools.partial(
        pl.kernel,
        mesh=mesh,
        out_type=jax.ShapeDtypeStruct((_T, 4, 32, 8, 128), jnp.float32),
        scratch_types=[
            pltpu.VMEM((_TBLK, _BBLK), jnp.int32),
            pltpu.VMEM((_H, _EMB), jnp.float32),
            pltpu.VMEM((_H, _EMB), jnp.float32),
            pltpu.VMEM((4, 4, 8, 128), jnp.float32),
            pltpu.VMEM((4, 4, 8, 128), jnp.float32),
            pltpu.SemaphoreType.DMA,
            pltpu.SemaphoreType.DMA,
            pltpu.SemaphoreType.DMA,
            pltpu.SemaphoreType.DMA,
        ],
        compiler_params=pltpu.CompilerParams(
            use_tc_tiling_on_sc=False, needs_layout_passes=False),
    )
    def emb_kernel(idsT_hbm, table_hbm, out_hbm, idx_v, ra, rb, sa, sb,
                   ga, gb, oa, ob):
        rows = [ra, rb]
        slab = [sa, sb]
        gsem = [ga, gb]
        osem = [oa, ob]
        wid = lax.axis_index("s") * _NC + lax.axis_index("c")
        tw = wid // _BPW
        bw = wid % _BPW
        t0 = tw * _TBLK
        b0 = bw * _BBLK
        bt0 = bw * 8

        pltpu.sync_copy(
            idsT_hbm.at[pl.ds(t0, _TBLK), pl.ds(b0, _BBLK)], idx_v)

        def idx_slice(step):
            return idx_v.at[step // 2, pl.ds((step % 2) * _H, _H)]

        def start_gather(step, p):
            pltpu.async_copy(table_hbm.at[idx_slice(step)], rows[p], gsem[p])

        def wait_gather(p):
            pltpu.make_async_copy(
                table_hbm.at[idx_slice(0)], rows[p], gsem[p]).wait()

        def wb_dst(step, et):
            return out_hbm.at[
                t0 + step // 2, et, pl.ds(bt0 + (step % 2) * 4, 4)]

        def start_wb(step, p):
            for et in range(4):
                pltpu.async_copy(slab[p].at[et], wb_dst(step, et), osem[p])

        def wait_wb(p):
            for et in range(4):
                pltpu.make_async_copy(
                    slab[p].at[et], wb_dst(0, et), osem[p]).wait()

        start_gather(0, 0)
        start_gather(1, 1)

        def pair(jj, carry):
            for p in range(2):
                step = 2 * jj + p
                wait_gather(p)

                @pl.when(jj >= 1)
                def _():
                    wait_wb(p)

                _transpose_rows_to_slab(rows[p], slab[p])

                @pl.when(step + 2 < _NSTEP)
                def _():
                    start_gather(step + 2, p)
                start_wb(step, p)
            return carry

        lax.fori_loop(0, _NSTEP // 2, pair, 0)
        wait_wb(0)
        wait_wb(1)

    return emb_kernel


def kernel(input_ids, weight):
    tail = weight[_NTC * 128:].reshape(-1)
    wlin = _make_prep_kernel()(weight.T, tail)
    table = wlin.reshape(_VOCAB, _EMB)
    idsT = input_ids.astype(jnp.int32).T
    buf = _make_gather_kernel()(idsT, table)
    return buf.transpose(2, 4, 0, 1, 3).reshape(_BATCH, _T, _EMB)


# speedup vs baseline: 2.4994x; 1.0431x over previous
import functools

import jax
import jax.numpy as jnp
from jax import lax
from jax.experimental import pallas as pl
from jax.experimental.pallas import tpu as pltpu
from jax.experimental.pallas import tpu_sc as plsc

_VOCAB = 1_000_000
_EMB = 32

_NC, _NS = 2, 16
_NW = _NC * _NS
_T = 200
_BATCH = 4096
_TPW = 8
_BPW = 4
_TBLK = _T // _TPW
_BBLK = _BATCH // _BPW
_H = 512
_NSTEP = _TBLK * 2

_NTC = _VOCAB // 128
_MAIN = 244


def _tile_transpose(t2, r1d, nkblk):
    iota = lax.iota(jnp.int32, 16)
    epre = [eh * 16 + ((iota + d) & 15)
            for eh in range(2) for d in range(16)]

    def body(k, carry):
        kv = k * 16 + iota
        kv32 = kv * 32
        for e in epre:
            vals = plsc.load_gather(t2, [e, kv])
            plsc.store_scatter(r1d, [kv32 + e], vals)
        return carry

    lax.fori_loop(0, nkblk, body, 0)


def _make_prep_kernel():
    mesh = plsc.VectorSubcoreMesh(core_axis_name="c", subcore_axis_name="s")

    @functools.partial(
        pl.kernel,
        mesh=mesh,
        out_type=jax.ShapeDtypeStruct((_VOCAB * _EMB,), jnp.float32),
        scratch_types=[
            pltpu.VMEM((32, 128), jnp.float32),
            pltpu.VMEM((32, 128), jnp.float32),
            pltpu.VMEM((4096,), jnp.float32),
            pltpu.VMEM((4096,), jnp.float32),
            pltpu.SemaphoreType.DMA,
            pltpu.SemaphoreType.DMA,
            pltpu.SemaphoreType.DMA,
            pltpu.SemaphoreType.DMA,
        ],
        compiler_params=pltpu.CompilerParams(
            use_tc_tiling_on_sc=True, needs_layout_passes=False),
    )
    def prep_kernel(wt_hbm, tail_hbm, out_hbm, ta, tb, ra, rb, la, lb, oa, ob):
        tiles = [ta, tb]
        rowb = [ra, rb]
        lsem = [la, lb]
        osem = [oa, ob]
        wid = lax.axis_index("s") * _NC + lax.axis_index("c")

        def load(j, p):
            c = wid + _NW * j
            pltpu.async_copy(
                wt_hbm.at[:, pl.ds(c * 128, 128)], tiles[p], lsem[p])

        def wait_load(p):
            pltpu.make_async_copy(
                wt_hbm.at[:, pl.ds(0, 128)], tiles[p], lsem[p]).wait()

        def wb(j, p):
            c = wid + _NW * j
            pltpu.async_copy(
                rowb[p], out_hbm.at[pl.ds(c * 4096, 4096)], osem[p])

        def wait_wb(p):
            pltpu.make_async_copy(
                rowb[p], out_hbm.at[pl.ds(0, 4096)], osem[p]).wait()

        load(0, 0)
        load(1, 1)

        def pair(jj, carry):
            for p in range(2):
                j = 2 * jj + p
                wait_load(p)

                @pl.when(jj >= 1)
                def _():
                    wait_wb(p)

                _tile_transpose(tiles[p], rowb[p], 8)

                @pl.when(j + 2 < _MAIN)
                def _():
                    load(j + 2, p)
                wb(j, p)
            return carry

        lax.fori_loop(0, _MAIN // 2, pair, 0)
        wait_wb(0)
        wait_wb(1)

        @pl.when(wid < 4)
        def _():
            c = _NW * _MAIN + wid
            pltpu.sync_copy(wt_hbm.at[:, pl.ds(c * 128, 128)], ta)
            _tile_transpose(ta, ra, 8)
            pltpu.sync_copy(ra, out_hbm.at[pl.ds(c * 4096, 4096)])

        @pl.when(wid == 4)
        def _():
            pltpu.sync_copy(tail_hbm, ra.at[pl.ds(0, 2048)])
            pltpu.sync_copy(
                ra.at[pl.ds(0, 2048)],
                out_hbm.at[pl.ds(_NTC * 4096, 2048)])

    return prep_kernel


def _transpose_rows_to_slab(rows_ref, slab_ref):
    iota = lax.iota(jnp.int32, 16)
    pre = []
    for ch in range(2):
        for d in range(16):
            perm = (iota + d) & 15
            cvec = ch * 16 + perm
            inv = (ch * 2 + (perm >> 3)) * 4096 + (perm & 7) * 128
            pre.append((cvec, inv))

    def body(bblk, carry):
        bvec = bblk * 16 + iota
        bv = (bblk // 8) * 1024 + (bblk % 8) * 16 + iota
        for cvec, inv in pre:
            vals = plsc.load_gather(rows_ref, [bvec, cvec])
            plsc.store_scatter(slab_ref, [inv + bv], vals)
        return carry

    lax.fori_loop(0, 32, body, 0)


def _make_gather_kernel():
    mesh = plsc.VectorSubcoreMesh(core_axis_name="c", subcore_axis_name="s")

    @functools.partial(
        pl.kernel,
        mesh=mesh,
        out_type=jax.ShapeDtypeStruct((_T * 4 * 32 * 8 * 128,), jnp.float32),
        scratch_types=[
            pltpu.VMEM((_TBLK, _BBLK), jnp.int32),
            pltpu.VMEM((_H, _EMB), jnp.float32),
            pltpu.VMEM((_H, _EMB), jnp.float32),
            pltpu.VMEM((16384,), jnp.float32),
            pltpu.VMEM((16384,), jnp.float32),
            pltpu.SemaphoreType.DMA,
            pltpu.SemaphoreType.DMA,
            pltpu.SemaphoreType.DMA,
            pltpu.SemaphoreType.DMA,
        ],
        compiler_params=pltpu.CompilerParams(
            use_tc_tiling_on_sc=False, needs_layout_passes=False),
    )
    def emb_kernel(idsT_hbm, table_hbm, out_hbm, idx_v, ra, rb, sa, sb,
                   ga, gb, oa, ob):
        rows = [ra, rb]
        slab = [sa, sb]
        gsem = [ga, gb]
        osem = [oa, ob]
        wid = lax.axis_index("s") * _NC + lax.axis_index("c")
        tw = wid // _BPW
        bw = wid % _BPW
        t0 = tw * _TBLK
        b0 = bw * _BBLK
        bt0 = bw * 8

        pltpu.sync_copy(
            idsT_hbm.at[pl.ds(t0, _TBLK), pl.ds(b0, _BBLK)], idx_v)

        def idx_slice(step):
            return idx_v.at[step // 2, pl.ds((step % 2) * _H, _H)]

        def start_gather(step, p):
            pltpu.async_copy(table_hbm.at[idx_slice(step)], rows[p], gsem[p])

        def wait_gather(p):
            pltpu.make_async_copy(
                table_hbm.at[idx_slice(0)], rows[p], gsem[p]).wait()

        def wb_dst(step, et):
            off = ((t0 + step // 2) * 4 + et) * 32768 \
                + (bt0 + (step % 2) * 4) * 1024
            return out_hbm.at[pl.ds(off, 4096)]

        def start_wb(step, p):
            for et in range(4):
                pltpu.async_copy(
                    slab[p].at[pl.ds(et * 4096, 4096)],
                    wb_dst(step, et), osem[p])

        def wait_wb(p):
            for et in range(4):
                pltpu.make_async_copy(
                    slab[p].at[pl.ds(0, 4096)], wb_dst(0, et),
                    osem[p]).wait()

        start_gather(0, 0)
        start_gather(1, 1)

        def pair(jj, carry):
            for p in range(2):
                step = 2 * jj + p
                wait_gather(p)

                @pl.when(jj >= 1)
                def _():
                    wait_wb(p)

                _transpose_rows_to_slab(rows[p], slab[p])

                @pl.when(step + 2 < _NSTEP)
                def _():
                    start_gather(step + 2, p)
                start_wb(step, p)
            return carry

        lax.fori_loop(0, _NSTEP // 2, pair, 0)
        wait_wb(0)
        wait_wb(1)

    return emb_kernel


def kernel(input_ids, weight):
    tail = weight[_NTC * 128:].reshape(-1)
    wlin = _make_prep_kernel()(weight.T, tail)
    table = wlin.reshape(_VOCAB, _EMB)
    idsT = input_ids.astype(jnp.int32).T
    buf = _make_gather_kernel()(idsT, table)
    return (buf.reshape(_T, 4, 32, 8, 128)
            .transpose(2, 4, 0, 1, 3).reshape(_BATCH, _T, _EMB))
